# Initial kernel scaffold; baseline (speedup 1.0000x reference)
import jax
import jax.numpy as jnp
from jax.experimental import pallas as pl


def kernel(z, embedding):
    raise NotImplementedError("write your pallas kernel here")


# speedup vs baseline: 4.4867x; 4.4867x over previous
import functools

import jax
import jax.numpy as jnp
from jax import lax
from jax.experimental import pallas as pl
from jax.experimental.pallas import tpu as pltpu
from jax.experimental.pallas import tpu_sc as plsc

D = 256
N = 8192
M = 8192
MB = 512
NB = 1024
NJ = N // NB
COMMITMENT_COST = 0.25


def _dist_kernel(z_ref, e_ref, idx_ref, loss_ref, et_ref, minv, mini):
    j = pl.program_id(0)
    i = pl.program_id(1)
    zb = z_ref[...]
    eb = e_ref[...]

    @pl.when(i == 0)
    def _():
        et_ref[...] = eb.T

    z2 = jnp.sum(zb * zb, axis=1, keepdims=True)
    e2 = jnp.sum(eb * eb, axis=0, keepdims=True)
    d = z2 - 2.0 * jnp.dot(zb, eb, preferred_element_type=jnp.float32) + e2
    m = jnp.min(d, axis=1)
    a = jnp.argmin(d, axis=1).astype(jnp.int32) + j * NB

    @pl.when(j == 0)
    def _():
        minv[i, :] = m
        mini[i, :] = a

    @pl.when(j > 0)
    def _():
        better = m < minv[i, :]
        mini[i, :] = jnp.where(better, a, mini[i, :])
        minv[i, :] = jnp.where(better, m, minv[i, :])

    @pl.when(j == NJ - 1)
    def _():
        idx_ref[...] = mini[i, :]
        part = jnp.sum(minv[i, :])

        @pl.when(i == 0)
        def _():
            loss_ref[...] = jnp.full((1, 1), part, jnp.float32)

        @pl.when(i > 0)
        def _():
            loss_ref[...] = loss_ref[...] + part


def _distance_argmin(z_flat, embedding):
    return pl.pallas_call(
        _dist_kernel,
        grid=(NJ, M // MB),
        in_specs=[
            pl.BlockSpec((MB, D), lambda j, i: (i, 0)),
            pl.BlockSpec((D, NB), lambda j, i: (0, j)),
        ],
        out_specs=[
            pl.BlockSpec((MB,), lambda j, i: (i,)),
            pl.BlockSpec((1, 1), lambda j, i: (0, 0)),
            pl.BlockSpec((NB, D), lambda j, i: (j, 0)),
        ],
        out_shape=[
            jax.ShapeDtypeStruct((M,), jnp.int32),
            jax.ShapeDtypeStruct((1, 1), jnp.float32),
            jax.ShapeDtypeStruct((N, D), jnp.float32),
        ],
        scratch_shapes=[
            pltpu.VMEM((M // MB, MB), jnp.float32),
            pltpu.VMEM((M // MB, MB), jnp.int32),
        ],
    )(z_flat, embedding)


_NW = 32
_BPW = M // _NW


def _sc_gather(table, idx):
    mesh = plsc.VectorSubcoreMesh(core_axis_name="c", subcore_axis_name="s")

    @functools.partial(
        pl.kernel,
        mesh=mesh,
        out_type=jax.ShapeDtypeStruct((M, D), jnp.float32),
        scratch_types=[
            pltpu.VMEM((_BPW,), jnp.int32),
            pltpu.VMEM((_BPW, D), jnp.float32),
            pltpu.SemaphoreType.DMA,
        ],
    )
    def k(table_hbm, idx_hbm, out_hbm, idx_v, rows_v, sem):
        wid = lax.axis_index("s") * 2 + lax.axis_index("c")
        base = wid * _BPW
        pltpu.sync_copy(idx_hbm.at[pl.ds(base, _BPW)], idx_v)
        pltpu.async_copy(table_hbm.at[idx_v], rows_v, sem).wait()
        pltpu.sync_copy(rows_v, out_hbm.at[pl.ds(base, _BPW)])

    return k(table, idx)


def kernel(z, embedding):
    z_flat = z.reshape(M, D)
    idx, loss_acc, table = _distance_argmin(z_flat, embedding)
    quantized = _sc_gather(table, idx).reshape(z.shape)
    loss = loss_acc[0, 0] * (COMMITMENT_COST / (M * D))
    return (quantized, loss)
